# baseline (speedup 1.0000x reference)
import math

import jax
import jax.numpy as jnp
from jax import lax
from jax.experimental import pallas as pl
from jax.experimental.pallas import tpu as pltpu
from jax.experimental.pallas import tpu_sc as plsc

N = 2048
SIZE = 32000
SMOOTHING = 0.1
EPS = SMOOTHING / (SIZE - 1)
CONF = 1.0 - SMOOTHING
C0 = (SIZE - 1) * EPS * math.log(EPS) + CONF * math.log(CONF)

NC = 2
NS = 16
LANES = 16
NW = NC * NS
BPW = N // NW

RSC = 1280
RPS = RSC // NW
NBUF = 2
UNROLL = 8
VECS = SIZE // (LANES * UNROLL)

SCOUT = N + NW * LANES

TCB = 32
NSTREAM = 4
NTC = N - RSC
RRS = NTC // NSTREAM
GRID = NTC // (NSTREAM * TCB)


def _sc_body(xflat, tgt, out, tgt_v, idx_v, val_v, buf0, buf1, stage,
             gsem, sem0, sem1):
    wid = lax.axis_index("s") * NC + lax.axis_index("c")
    row0 = wid * RPS
    bufs = (buf0, buf1)
    sems = (sem0, sem1)
    for b in range(NBUF):
        pltpu.async_copy(
            xflat.at[pl.ds((row0 + b) * SIZE, SIZE)], bufs[b], sems[b]
        )

    base = wid * BPW
    pltpu.sync_copy(tgt.at[pl.ds(base, BPW)], tgt_v)
    for j in range(BPW // LANES):
        rows = (base + j * LANES) + lax.broadcasted_iota(jnp.int32, (LANES,), 0)
        idx_v[pl.ds(j * LANES, LANES)] = rows * SIZE + tgt_v[pl.ds(j * LANES, LANES)]
    pltpu.async_copy(xflat.at[idx_v], val_v, gsem).wait()
    pltpu.sync_copy(val_v, out.at[pl.ds(base, BPW)])

    accs = tuple(jnp.zeros((LANES,), jnp.float32) for _ in range(UNROLL))

    def row_group(g, accs):
        for b in range(NBUF):
            r = g * NBUF + b
            cur = bufs[b]
            pltpu.make_async_copy(
                xflat.at[pl.ds(0, SIZE)], cur, sems[b]
            ).wait()

            def body(i, accs_, cur=cur):
                base = i * (LANES * UNROLL)
                return tuple(
                    a + cur[pl.ds(base + k * LANES, LANES)]
                    for k, a in enumerate(accs_)
                )

            accs = lax.fori_loop(0, VECS, body, accs)

            @pl.when(r + NBUF < RPS)
            def _refill(r=r, b=b):
                pltpu.async_copy(
                    xflat.at[pl.ds((row0 + r + NBUF) * SIZE, SIZE)],
                    bufs[b],
                    sems[b],
                )

        return accs

    accs = lax.fori_loop(0, RPS // NBUF, row_group, accs)
    total = accs[0]
    for a in accs[1:]:
        total = total + a
    stage[...] = total
    pltpu.sync_copy(stage, out.at[pl.ds(N + wid * LANES, LANES)])


def _sc_stage(xflat, tgt):
    k = pl.kernel(
        _sc_body,
        out_type=jax.ShapeDtypeStruct((SCOUT,), jnp.float32),
        mesh=plsc.VectorSubcoreMesh(core_axis_name="c", subcore_axis_name="s"),
        scratch_types=[
            pltpu.VMEM((BPW,), jnp.int32),
            pltpu.VMEM((BPW,), jnp.int32),
            pltpu.VMEM((BPW,), jnp.float32),
            pltpu.VMEM((SIZE,), jnp.float32),
            pltpu.VMEM((SIZE,), jnp.float32),
            pltpu.VMEM((LANES,), jnp.float32),
            pltpu.SemaphoreType.DMA,
            pltpu.SemaphoreType.DMA,
            pltpu.SemaphoreType.DMA,
        ],
    )
    return k(xflat, tgt)


def _tc_sum_body(x0, x1, x2, x3, out, acc):
    i = pl.program_id(0)

    @pl.when(i == 0)
    def _init():
        acc[0] = jnp.float32(0.0)

    acc[0] += (
        jnp.sum(x0[...]) + jnp.sum(x1[...]) + jnp.sum(x2[...]) + jnp.sum(x3[...])
    )

    @pl.when(i == GRID - 1)
    def _finish():
        out[0, 0] = acc[0]


def _tc_sum(x):
    stream_specs = [
        pl.BlockSpec(
            (TCB, SIZE),
            lambda i, j=j: ((RSC + j * RRS) // TCB + i, 0),
        )
        for j in range(NSTREAM)
    ]
    out = pl.pallas_call(
        _tc_sum_body,
        grid=(GRID,),
        in_specs=stream_specs,
        out_specs=pl.BlockSpec(memory_space=pltpu.SMEM),
        out_shape=jax.ShapeDtypeStruct((1, 1), jnp.float32),
        scratch_shapes=[pltpu.SMEM((1,), jnp.float32)],
    )(x, x, x, x)
    return out


def _combine_body(sc_ref, tcp_ref, out_ref):
    g = jnp.sum(sc_ref[0 : N // 128, :])
    s = jnp.sum(sc_ref[N // 128 :, :]) + tcp_ref[0, 0]
    out_ref[0, 0] = (
        jnp.float32(C0)
        - jnp.float32(EPS) * (s / N)
        + jnp.float32(EPS - CONF) * (g / N)
    )


def _combine(scout, tcp):
    out = pl.pallas_call(
        _combine_body,
        in_specs=[
            pl.BlockSpec((SCOUT // 128, 128), lambda: (0, 0)),
            pl.BlockSpec(memory_space=pltpu.SMEM),
        ],
        out_specs=pl.BlockSpec(memory_space=pltpu.SMEM),
        out_shape=jax.ShapeDtypeStruct((1, 1), jnp.float32),
    )(scout.reshape(SCOUT // 128, 128), tcp)
    return out[0, 0]


def kernel(x, target):
    tgt = target.astype(jnp.int32)
    xflat = x.reshape(N * SIZE)
    scout = _sc_stage(xflat, tgt)
    tcp = _tc_sum(x)
    return _combine(scout, tcp)


# speedup vs baseline: 1.0235x; 1.0235x over previous
import math

import jax
import jax.numpy as jnp
from jax import lax
from jax.experimental import pallas as pl
from jax.experimental.pallas import tpu as pltpu
from jax.experimental.pallas import tpu_sc as plsc

N = 2048
SIZE = 32000
SMOOTHING = 0.1
EPS = SMOOTHING / (SIZE - 1)
CONF = 1.0 - SMOOTHING
C0 = (SIZE - 1) * EPS * math.log(EPS) + CONF * math.log(CONF)

NC = 2
NS = 16
LANES = 16
NW = NC * NS
BPW = N // NW

RSC = 768
RPS = RSC // NW
NBUF = 2
UNROLL = 8
VECS = SIZE // (LANES * UNROLL)

SCOUT = N + NW * LANES

TCB = 32
NSTREAM = 4
NTC = N - RSC
RRS = NTC // NSTREAM
GRID = NTC // (NSTREAM * TCB)


def _sc_body(xflat, tgt, out, tgt_v, idx_v, val_v, buf0, buf1, stage,
             gsem, sem0, sem1):
    wid = lax.axis_index("s") * NC + lax.axis_index("c")
    row0 = wid * RPS
    bufs = (buf0, buf1)
    sems = (sem0, sem1)
    for b in range(NBUF):
        pltpu.async_copy(
            xflat.at[pl.ds((row0 + b) * SIZE, SIZE)], bufs[b], sems[b]
        )

    base = wid * BPW
    pltpu.sync_copy(tgt.at[pl.ds(base, BPW)], tgt_v)
    for j in range(BPW // LANES):
        rows = (base + j * LANES) + lax.broadcasted_iota(jnp.int32, (LANES,), 0)
        idx_v[pl.ds(j * LANES, LANES)] = rows * SIZE + tgt_v[pl.ds(j * LANES, LANES)]
    pltpu.async_copy(xflat.at[idx_v], val_v, gsem).wait()
    pltpu.sync_copy(val_v, out.at[pl.ds(base, BPW)])

    accs = tuple(jnp.zeros((LANES,), jnp.float32) for _ in range(UNROLL))

    def row_group(g, accs):
        for b in range(NBUF):
            r = g * NBUF + b
            cur = bufs[b]
            pltpu.make_async_copy(
                xflat.at[pl.ds(0, SIZE)], cur, sems[b]
            ).wait()

            def body(i, accs_, cur=cur):
                base = i * (LANES * UNROLL)
                return tuple(
                    a + cur[pl.ds(base + k * LANES, LANES)]
                    for k, a in enumerate(accs_)
                )

            accs = lax.fori_loop(0, VECS, body, accs)

            @pl.when(r + NBUF < RPS)
            def _refill(r=r, b=b):
                pltpu.async_copy(
                    xflat.at[pl.ds((row0 + r + NBUF) * SIZE, SIZE)],
                    bufs[b],
                    sems[b],
                )

        return accs

    accs = lax.fori_loop(0, RPS // NBUF, row_group, accs)
    total = accs[0]
    for a in accs[1:]:
        total = total + a
    stage[...] = total
    pltpu.sync_copy(stage, out.at[pl.ds(N + wid * LANES, LANES)])


def _sc_stage(xflat, tgt):
    k = pl.kernel(
        _sc_body,
        out_type=jax.ShapeDtypeStruct((SCOUT,), jnp.float32),
        mesh=plsc.VectorSubcoreMesh(core_axis_name="c", subcore_axis_name="s"),
        scratch_types=[
            pltpu.VMEM((BPW,), jnp.int32),
            pltpu.VMEM((BPW,), jnp.int32),
            pltpu.VMEM((BPW,), jnp.float32),
            pltpu.VMEM((SIZE,), jnp.float32),
            pltpu.VMEM((SIZE,), jnp.float32),
            pltpu.VMEM((LANES,), jnp.float32),
            pltpu.SemaphoreType.DMA,
            pltpu.SemaphoreType.DMA,
            pltpu.SemaphoreType.DMA,
        ],
    )
    return k(xflat, tgt)


def _tc_sum_body(x0, x1, x2, x3, out, acc):
    i = pl.program_id(0)

    @pl.when(i == 0)
    def _init():
        acc[0] = jnp.float32(0.0)

    acc[0] += (
        jnp.sum(x0[...]) + jnp.sum(x1[...]) + jnp.sum(x2[...]) + jnp.sum(x3[...])
    )

    @pl.when(i == GRID - 1)
    def _finish():
        out[0, 0] = acc[0]


def _tc_sum(x):
    stream_specs = [
        pl.BlockSpec(
            (TCB, SIZE),
            lambda i, j=j: ((RSC + j * RRS) // TCB + i, 0),
        )
        for j in range(NSTREAM)
    ]
    out = pl.pallas_call(
        _tc_sum_body,
        grid=(GRID,),
        in_specs=stream_specs,
        out_specs=pl.BlockSpec(memory_space=pltpu.SMEM),
        out_shape=jax.ShapeDtypeStruct((1, 1), jnp.float32),
        scratch_shapes=[pltpu.SMEM((1,), jnp.float32)],
    )(x, x, x, x)
    return out


def _combine_body(sc_ref, tcp_ref, out_ref):
    g = jnp.sum(sc_ref[0 : N // 128, :])
    s = jnp.sum(sc_ref[N // 128 :, :]) + tcp_ref[0, 0]
    out_ref[0, 0] = (
        jnp.float32(C0)
        - jnp.float32(EPS) * (s / N)
        + jnp.float32(EPS - CONF) * (g / N)
    )


def _combine(scout, tcp):
    out = pl.pallas_call(
        _combine_body,
        in_specs=[
            pl.BlockSpec((SCOUT // 128, 128), lambda: (0, 0)),
            pl.BlockSpec(memory_space=pltpu.SMEM),
        ],
        out_specs=pl.BlockSpec(memory_space=pltpu.SMEM),
        out_shape=jax.ShapeDtypeStruct((1, 1), jnp.float32),
    )(scout.reshape(SCOUT // 128, 128), tcp)
    return out[0, 0]


def kernel(x, target):
    tgt = target.astype(jnp.int32)
    xflat = x.reshape(N * SIZE)
    scout = _sc_stage(xflat, tgt)
    tcp = _tc_sum(x)
    return _combine(scout, tcp)


# speedup vs baseline: 1.0270x; 1.0034x over previous
import math

import jax
import jax.numpy as jnp
from jax import lax
from jax.experimental import pallas as pl
from jax.experimental.pallas import tpu as pltpu
from jax.experimental.pallas import tpu_sc as plsc

N = 2048
SIZE = 32000
SMOOTHING = 0.1
EPS = SMOOTHING / (SIZE - 1)
CONF = 1.0 - SMOOTHING
C0 = (SIZE - 1) * EPS * math.log(EPS) + CONF * math.log(CONF)

NC = 2
NS = 16
LANES = 16
NW = NC * NS
BPW = N // NW

RSC = 512
RPS = RSC // NW
NBUF = 2
UNROLL = 8
VECS = SIZE // (LANES * UNROLL)

SCOUT = N + NW * LANES

TCB = 32
NSTREAM = 4
NTC = N - RSC
RRS = NTC // NSTREAM
GRID = NTC // (NSTREAM * TCB)


def _sc_body(xflat, tgt, out, tgt_v, idx_v, val_v, buf0, buf1, stage,
             gsem, sem0, sem1):
    wid = lax.axis_index("s") * NC + lax.axis_index("c")
    row0 = wid * RPS
    bufs = (buf0, buf1)
    sems = (sem0, sem1)
    for b in range(NBUF):
        pltpu.async_copy(
            xflat.at[pl.ds((row0 + b) * SIZE, SIZE)], bufs[b], sems[b]
        )

    base = wid * BPW
    pltpu.sync_copy(tgt.at[pl.ds(base, BPW)], tgt_v)
    for j in range(BPW // LANES):
        rows = (base + j * LANES) + lax.broadcasted_iota(jnp.int32, (LANES,), 0)
        idx_v[pl.ds(j * LANES, LANES)] = rows * SIZE + tgt_v[pl.ds(j * LANES, LANES)]
    pltpu.async_copy(xflat.at[idx_v], val_v, gsem).wait()
    pltpu.sync_copy(val_v, out.at[pl.ds(base, BPW)])

    accs = tuple(jnp.zeros((LANES,), jnp.float32) for _ in range(UNROLL))

    def row_group(g, accs):
        for b in range(NBUF):
            r = g * NBUF + b
            cur = bufs[b]
            pltpu.make_async_copy(
                xflat.at[pl.ds(0, SIZE)], cur, sems[b]
            ).wait()

            def body(i, accs_, cur=cur):
                base = i * (LANES * UNROLL)
                return tuple(
                    a + cur[pl.ds(base + k * LANES, LANES)]
                    for k, a in enumerate(accs_)
                )

            accs = lax.fori_loop(0, VECS, body, accs)

            @pl.when(r + NBUF < RPS)
            def _refill(r=r, b=b):
                pltpu.async_copy(
                    xflat.at[pl.ds((row0 + r + NBUF) * SIZE, SIZE)],
                    bufs[b],
                    sems[b],
                )

        return accs

    accs = lax.fori_loop(0, RPS // NBUF, row_group, accs)
    total = accs[0]
    for a in accs[1:]:
        total = total + a
    stage[...] = total
    pltpu.sync_copy(stage, out.at[pl.ds(N + wid * LANES, LANES)])


def _sc_stage(xflat, tgt):
    k = pl.kernel(
        _sc_body,
        out_type=jax.ShapeDtypeStruct((SCOUT,), jnp.float32),
        mesh=plsc.VectorSubcoreMesh(core_axis_name="c", subcore_axis_name="s"),
        scratch_types=[
            pltpu.VMEM((BPW,), jnp.int32),
            pltpu.VMEM((BPW,), jnp.int32),
            pltpu.VMEM((BPW,), jnp.float32),
            pltpu.VMEM((SIZE,), jnp.float32),
            pltpu.VMEM((SIZE,), jnp.float32),
            pltpu.VMEM((LANES,), jnp.float32),
            pltpu.SemaphoreType.DMA,
            pltpu.SemaphoreType.DMA,
            pltpu.SemaphoreType.DMA,
        ],
    )
    return k(xflat, tgt)


def _tc_sum_body(x0, x1, x2, x3, out, acc):
    i = pl.program_id(0)

    @pl.when(i == 0)
    def _init():
        acc[0] = jnp.float32(0.0)

    acc[0] += (
        jnp.sum(x0[...]) + jnp.sum(x1[...]) + jnp.sum(x2[...]) + jnp.sum(x3[...])
    )

    @pl.when(i == GRID - 1)
    def _finish():
        out[0, 0] = acc[0]


def _tc_sum(x):
    stream_specs = [
        pl.BlockSpec(
            (TCB, SIZE),
            lambda i, j=j: ((RSC + j * RRS) // TCB + i, 0),
        )
        for j in range(NSTREAM)
    ]
    out = pl.pallas_call(
        _tc_sum_body,
        grid=(GRID,),
        in_specs=stream_specs,
        out_specs=pl.BlockSpec(memory_space=pltpu.SMEM),
        out_shape=jax.ShapeDtypeStruct((1, 1), jnp.float32),
        scratch_shapes=[pltpu.SMEM((1,), jnp.float32)],
    )(x, x, x, x)
    return out


def _combine_body(sc_ref, tcp_ref, out_ref):
    g = jnp.sum(sc_ref[0 : N // 128, :])
    s = jnp.sum(sc_ref[N // 128 :, :]) + tcp_ref[0, 0]
    out_ref[0, 0] = (
        jnp.float32(C0)
        - jnp.float32(EPS) * (s / N)
        + jnp.float32(EPS - CONF) * (g / N)
    )


def _combine(scout, tcp):
    out = pl.pallas_call(
        _combine_body,
        in_specs=[
            pl.BlockSpec((SCOUT // 128, 128), lambda: (0, 0)),
            pl.BlockSpec(memory_space=pltpu.SMEM),
        ],
        out_specs=pl.BlockSpec(memory_space=pltpu.SMEM),
        out_shape=jax.ShapeDtypeStruct((1, 1), jnp.float32),
    )(scout.reshape(SCOUT // 128, 128), tcp)
    return out[0, 0]


def kernel(x, target):
    tgt = target.astype(jnp.int32)
    xflat = x.reshape(N * SIZE)
    scout = _sc_stage(xflat, tgt)
    tcp = _tc_sum(x)
    return _combine(scout, tcp)
